# baseline (speedup 1.0000x reference)
import functools

import jax
import jax.numpy as jnp
from jax import lax
from jax.experimental import pallas as pl
from jax.experimental.pallas import tpu as pltpu
from jax.experimental.pallas import tpu_sc as plsc

B = 16384
V = 100000
D_ART = 64
D_ROW = 128
EPS = 1e-3

_NC = 2
_NS = 16
NW = _NC * _NS
BPW = B // NW
IDX_CHUNK = 128
NCHUNK = BPW // IDX_CHUNK

_sc_mesh = plsc.VectorSubcoreMesh(
    core_axis_name="c", subcore_axis_name="s", num_cores=_NC, num_subcores=_NS)


@functools.partial(
    pl.kernel,
    out_type=jax.ShapeDtypeStruct((B, 128), jnp.float32),
    mesh=_sc_mesh,
    scratch_types=(
        pltpu.VMEM((NCHUNK, IDX_CHUNK), jnp.int32),
        pltpu.VMEM((BPW, D_ROW), jnp.float32),
        pltpu.SemaphoreType.DMA,
    ),
    compiler_params=pltpu.CompilerParams(use_tc_tiling_on_sc=False),
)
def _sc_gather(ids_hbm, emb_hbm, out_hbm, idx_v, rows_v, sem):
    wid = lax.axis_index("s") * _NC + lax.axis_index("c")
    base = wid * BPW
    pltpu.sync_copy(ids_hbm.at[pl.ds(wid * NCHUNK, NCHUNK)], idx_v)
    copies = []
    for j in range(NCHUNK):
        sl = pl.ds(j * IDX_CHUNK, IDX_CHUNK)
        copies.append(pltpu.async_copy(emb_hbm.at[idx_v.at[j]], rows_v.at[sl], sem))
    for cp in copies:
        cp.wait()
    pltpu.sync_copy(rows_v, out_hbm.at[pl.ds(base, BPW), pl.ds(0, D_ROW)])


BLK = 2048


def _tc_body(art_ref, semb_ref, gemb_ref, gremb_ref, cemb_ref,
             gamma_ref, beta_ref, mean_ref, var_ref, w_ref, out_ref):
    scale = gamma_ref[:] * lax.rsqrt(var_ref[:] + EPS)
    shift = beta_ref[:] - mean_ref[:] * scale
    packed = lax.bitcast_convert_type(
        art_ref[:, D_ART:D_ART + 1], jnp.int32)

    def onehot_feat(idx, emb_ref, ncls):
        oh = (idx == lax.broadcasted_iota(jnp.int32, (1, ncls), 1))
        return jnp.dot(oh.astype(jnp.float32), emb_ref[:],
                       preferred_element_type=jnp.float32)

    xs = onehot_feat(packed & 63, semb_ref, 64)
    xg = onehot_feat((packed >> 6) & 31, gemb_ref, 32)
    xgr = onehot_feat((packed >> 11) & 31, gremb_ref, 32)
    xc = onehot_feat((packed >> 16) & 31, cemb_ref, 32)
    x = jnp.concatenate([art_ref[:, :D_ART], xg, xgr, xc, xs], axis=1)
    x = x * scale + shift
    out_ref[:] = jnp.dot(x, w_ref[:], preferred_element_type=jnp.float32)


def _tc_dense(art, semb, gemb, gremb, cemb, gamma, beta, mean, var, w):
    grid = (B // BLK,)
    full = lambda a: pl.BlockSpec(a.shape, lambda i: tuple(0 for _ in a.shape))
    return pl.pallas_call(
        _tc_body,
        grid=grid,
        in_specs=[
            pl.BlockSpec((BLK, 128), lambda i: (i, 0)),
            full(semb), full(gemb), full(gremb), full(cemb),
            full(gamma), full(beta), full(mean), full(var), full(w),
        ],
        out_specs=pl.BlockSpec((BLK, 128), lambda i: (i, 0)),
        out_shape=jax.ShapeDtypeStruct((B, 128), jnp.float32),
    )(art, semb, gemb, gremb, cemb, gamma, beta, mean, var, w)


def kernel(article_id, article_emb, section_map, section_emb, group_map,
           group_emb, graphical_map, graphical_emb, colour_map, colour_emb,
           gamma, beta, moving_mean, moving_var, W):
    ids = article_id.astype(jnp.int32).reshape(B // IDX_CHUNK, IDX_CHUNK)
    packed = (section_map | (group_map << 6) | (graphical_map << 11)
              | (colour_map << 16) | 0x40000000).astype(jnp.int32)
    pk_f32 = lax.bitcast_convert_type(packed, jnp.float32)
    emb80 = jnp.concatenate(
        [article_emb, pk_f32[:, None],
         jnp.zeros((V, D_ROW - D_ART - 1), jnp.float32)], axis=1)
    art = _sc_gather(ids, emb80)
    return _tc_dense(
        art, section_emb, group_emb, graphical_emb, colour_emb,
        gamma.reshape(1, 128), beta.reshape(1, 128),
        moving_mean.reshape(1, 128), moving_var.reshape(1, 128), W)


# speedup vs baseline: 1.0861x; 1.0861x over previous
import functools

import jax
import jax.numpy as jnp
from jax import lax
from jax.experimental import pallas as pl
from jax.experimental.pallas import tpu as pltpu
from jax.experimental.pallas import tpu_sc as plsc

B = 16384
V = 100000
D_ART = 64
EPS = 1e-3
LANES = 16

_NC = 2
_NS = 16
NW = _NC * _NS
BPW = B // NW
IDX_CHUNK = 128
NCHUNK = BPW // IDX_CHUNK
NVREG = BPW // LANES

_sc_mesh = plsc.VectorSubcoreMesh(
    core_axis_name="c", subcore_axis_name="s", num_cores=_NC, num_subcores=_NS)


PMAP_ROWS = (V + 127) // 128


@functools.partial(
    pl.kernel,
    out_type=jax.ShapeDtypeStruct((B, 128), jnp.float32),
    mesh=_sc_mesh,
    scratch_types=(
        pltpu.VMEM((NCHUNK, IDX_CHUNK), jnp.int32),
        pltpu.VMEM((NCHUNK, IDX_CHUNK), jnp.int32),
        pltpu.VMEM((BPW, D_ART), jnp.float32),
        pltpu.VMEM((BPW, 128), jnp.int32),
        pltpu.VMEM((BPW, 4), jnp.float32),
        pltpu.SemaphoreType.DMA,
    ),
    compiler_params=pltpu.CompilerParams(
        use_tc_tiling_on_sc=False, needs_layout_passes=False),
)
def _sc_gather(ids_hbm, emb_hbm, pmap_hbm, out_hbm,
               idx_v, idx7_v, rows_v, mp_v, sidx_v, sem):
    wid = lax.axis_index("s") * _NC + lax.axis_index("c")
    base = wid * BPW
    pltpu.sync_copy(ids_hbm.at[pl.ds(wid * NCHUNK, NCHUNK)], idx_v)
    for k in range(NVREG):
        j, off = k // 8, (k % 8) * LANES
        v = idx_v[j, pl.ds(off, LANES)]
        idx7_v[j, pl.ds(off, LANES)] = lax.shift_right_logical(v, 7)
    copies = []
    for j in range(NCHUNK):
        sl = pl.ds(j * IDX_CHUNK, IDX_CHUNK)
        copies.append(pltpu.async_copy(emb_hbm.at[idx_v.at[j]], rows_v.at[sl], sem))
        copies.append(pltpu.async_copy(pmap_hbm.at[idx7_v.at[j]], mp_v.at[sl], sem))
    for cp in copies:
        cp.wait()
    iota = lax.iota(jnp.int32, LANES)
    zero = jnp.full((LANES,), 0, jnp.int32)
    for k in range(NVREG):
        j, off = k // 8, (k % 8) * LANES
        lanes = jnp.bitwise_and(idx_v[j, pl.ds(off, LANES)], 127)
        row_ids = iota + (k * LANES)
        val = plsc.load_gather(mp_v, [row_ids, lanes])
        plsc.store_scatter(sidx_v, [row_ids, zero],
                           plsc.bitcast(val, jnp.float32))
    out_rows = out_hbm.at[pl.ds(base, BPW)]
    pltpu.sync_copy(rows_v, out_rows.at[:, pl.ds(0, D_ART)])
    pltpu.sync_copy(sidx_v, out_rows.at[:, pl.ds(D_ART, 4)])


BLK = 1024


def _tc_body(art_ref, semb_ref, gemb_ref, gremb_ref, cemb_ref,
             gamma_ref, beta_ref, mean_ref, var_ref, w_ref, out_ref):
    scale = gamma_ref[:] * lax.rsqrt(var_ref[:] + EPS)
    shift = beta_ref[:] - mean_ref[:] * scale

    packed = lax.bitcast_convert_type(
        art_ref[:, D_ART:D_ART + 1], jnp.int32)

    def onehot_feat(idx, emb_ref, ncls):
        oh = (idx == lax.broadcasted_iota(jnp.int32, (1, ncls), 1))
        return jnp.dot(oh.astype(jnp.float32), emb_ref[:],
                       preferred_element_type=jnp.float32)

    xs = onehot_feat(packed & 63, semb_ref, 64)
    xg = onehot_feat((packed >> 6) & 31, gemb_ref, 32)
    xgr = onehot_feat((packed >> 11) & 31, gremb_ref, 32)
    xc = onehot_feat((packed >> 16) & 31, cemb_ref, 32)
    x = jnp.concatenate([art_ref[:, :D_ART], xg, xgr, xc, xs], axis=1)
    x = x * scale + shift
    out_ref[:] = jnp.dot(x, w_ref[:], preferred_element_type=jnp.float32)


def _tc_dense(art, semb, gemb, gremb, cemb, gamma, beta, mean, var, w):
    grid = (B // BLK,)
    full = lambda a: pl.BlockSpec(a.shape, lambda i: tuple(0 for _ in a.shape))
    return pl.pallas_call(
        _tc_body,
        grid=grid,
        in_specs=[
            pl.BlockSpec((BLK, 128), lambda i: (i, 0)),
            full(semb), full(gemb), full(gremb), full(cemb),
            full(gamma), full(beta), full(mean), full(var), full(w),
        ],
        out_specs=pl.BlockSpec((BLK, 128), lambda i: (i, 0)),
        out_shape=jax.ShapeDtypeStruct((B, 128), jnp.float32),
    )(art, semb, gemb, gremb, cemb, gamma, beta, mean, var, w)


def kernel(article_id, article_emb, section_map, section_emb, group_map,
           group_emb, graphical_map, graphical_emb, colour_map, colour_emb,
           gamma, beta, moving_mean, moving_var, W):
    ids = article_id.astype(jnp.int32).reshape(B // IDX_CHUNK, IDX_CHUNK)
    packed = (section_map | (group_map << 6) | (graphical_map << 11)
              | (colour_map << 16)).astype(jnp.int32)
    pmap = jnp.pad(packed, (0, PMAP_ROWS * 128 - V)).reshape(PMAP_ROWS, 128)
    art = _sc_gather(ids, article_emb, pmap)
    return _tc_dense(
        art, section_emb, group_emb, graphical_emb, colour_emb,
        gamma.reshape(1, 128), beta.reshape(1, 128),
        moving_mean.reshape(1, 128), moving_var.reshape(1, 128), W)


# speedup vs baseline: 1.2613x; 1.1614x over previous
import functools

import jax
import jax.numpy as jnp
from jax import lax
from jax.experimental import pallas as pl
from jax.experimental.pallas import tpu as pltpu
from jax.experimental.pallas import tpu_sc as plsc

B = 16384
V = 100000
D_ART = 64
EPS = 1e-3
LANES = 16

_NC = 2
_NS = 16
NW = _NC * _NS
BPW = B // NW
IDX_CHUNK = 128
NCHUNK = BPW // IDX_CHUNK
NVREG = BPW // LANES
PMAP_ROWS = (V + 127) // 128

_sc_mesh = plsc.VectorSubcoreMesh(
    core_axis_name="c", subcore_axis_name="s", num_cores=_NC, num_subcores=_NS)

_sc_params = pltpu.CompilerParams(
    use_tc_tiling_on_sc=False, needs_layout_passes=False)


@functools.partial(
    pl.kernel,
    out_type=jax.ShapeDtypeStruct((B, 128), jnp.float32),
    mesh=_sc_mesh,
    scratch_types=(
        pltpu.VMEM((NCHUNK, IDX_CHUNK), jnp.int32),
        pltpu.VMEM((NCHUNK, IDX_CHUNK), jnp.int32),
        pltpu.VMEM((BPW, 128), jnp.int32),
        pltpu.VMEM((BPW, 4), jnp.float32),
        pltpu.SemaphoreType.DMA,
    ),
    compiler_params=_sc_params,
)
def _sc_maps(ids_hbm, pmap_hbm, out_hbm, idx_v, idx7_v, mp_v, sidx_v, sem):
    wid = lax.axis_index("s") * _NC + lax.axis_index("c")
    base = wid * BPW
    pltpu.sync_copy(ids_hbm.at[pl.ds(wid * NCHUNK, NCHUNK)], idx_v)
    for k in range(NVREG):
        j, off = k // 8, (k % 8) * LANES
        v = idx_v[j, pl.ds(off, LANES)]
        idx7_v[j, pl.ds(off, LANES)] = lax.shift_right_logical(v, 7)
    copies = []
    for j in range(NCHUNK):
        sl = pl.ds(j * IDX_CHUNK, IDX_CHUNK)
        copies.append(pltpu.async_copy(pmap_hbm.at[idx7_v.at[j]], mp_v.at[sl], sem))
    for cp in copies:
        cp.wait()
    iota = lax.iota(jnp.int32, LANES)
    zero = jnp.full((LANES,), 0, jnp.int32)
    for k in range(NVREG):
        j, off = k // 8, (k % 8) * LANES
        lanes = jnp.bitwise_and(idx_v[j, pl.ds(off, LANES)], 127)
        row_ids = iota + (k * LANES)
        val = plsc.load_gather(mp_v, [row_ids, lanes])
        plsc.store_scatter(sidx_v, [row_ids, zero],
                           plsc.bitcast(val, jnp.float32))
    pltpu.sync_copy(sidx_v, out_hbm.at[pl.ds(base, BPW), pl.ds(0, 4)])


@functools.partial(
    pl.kernel,
    out_type=jax.ShapeDtypeStruct((B, 128), jnp.float32),
    mesh=_sc_mesh,
    scratch_types=(
        pltpu.VMEM((NCHUNK, IDX_CHUNK), jnp.int32),
        pltpu.VMEM((BPW, D_ART), jnp.float32),
        pltpu.SemaphoreType.DMA,
    ),
    compiler_params=_sc_params,
)
def _sc_article(ids_hbm, emb_hbm, out_hbm, idx_v, rows_v, sem):
    wid = lax.axis_index("s") * _NC + lax.axis_index("c")
    base = wid * BPW
    pltpu.sync_copy(ids_hbm.at[pl.ds(wid * NCHUNK, NCHUNK)], idx_v)
    copies = []
    for j in range(NCHUNK):
        sl = pl.ds(j * IDX_CHUNK, IDX_CHUNK)
        copies.append(pltpu.async_copy(emb_hbm.at[idx_v.at[j]], rows_v.at[sl], sem))
    for cp in copies:
        cp.wait()
    pltpu.sync_copy(rows_v, out_hbm.at[pl.ds(base, BPW), pl.ds(0, D_ART)])


BLK = 2048


def _tc_body(art_ref, sidx_ref, semb_ref, gemb_ref, gremb_ref, cemb_ref,
             gamma_ref, beta_ref, mean_ref, var_ref, w_ref, out_ref):
    scale = gamma_ref[:] * lax.rsqrt(var_ref[:] + EPS)
    shift = beta_ref[:] - mean_ref[:] * scale
    packed = lax.bitcast_convert_type(sidx_ref[:, 0:1], jnp.int32)

    def onehot_feat(idx, emb_ref, ncls):
        oh = (idx == lax.broadcasted_iota(jnp.int32, (1, ncls), 1))
        return jnp.dot(oh.astype(jnp.float32), emb_ref[:],
                       preferred_element_type=jnp.float32)

    xs = onehot_feat(packed & 63, semb_ref, 64)
    xg = onehot_feat((packed >> 6) & 31, gemb_ref, 32)
    xgr = onehot_feat((packed >> 11) & 31, gremb_ref, 32)
    xc = onehot_feat((packed >> 16) & 31, cemb_ref, 32)
    x = jnp.concatenate([art_ref[:, :D_ART], xg, xgr, xc, xs], axis=1)
    x = x * scale + shift
    out_ref[:] = jnp.dot(x, w_ref[:], preferred_element_type=jnp.float32)


def _tc_dense(art, sidx, semb, gemb, gremb, cemb, gamma, beta, mean, var, w):
    grid = (B // BLK,)
    full = lambda a: pl.BlockSpec(a.shape, lambda i: tuple(0 for _ in a.shape))
    row_blk = pl.BlockSpec((BLK, 128), lambda i: (i, 0))
    return pl.pallas_call(
        _tc_body,
        grid=grid,
        in_specs=[
            row_blk, row_blk,
            full(semb), full(gemb), full(gremb), full(cemb),
            full(gamma), full(beta), full(mean), full(var), full(w),
        ],
        out_specs=row_blk,
        out_shape=jax.ShapeDtypeStruct((B, 128), jnp.float32),
    )(art, sidx, semb, gemb, gremb, cemb, gamma, beta, mean, var, w)


def kernel(article_id, article_emb, section_map, section_emb, group_map,
           group_emb, graphical_map, graphical_emb, colour_map, colour_emb,
           gamma, beta, moving_mean, moving_var, W):
    ids = article_id.astype(jnp.int32).reshape(B // IDX_CHUNK, IDX_CHUNK)
    packed = (section_map | (group_map << 6) | (graphical_map << 11)
              | (colour_map << 16)).astype(jnp.int32)
    pmap = jnp.pad(packed, (0, PMAP_ROWS * 128 - V)).reshape(PMAP_ROWS, 128)
    sidx = _sc_maps(ids, pmap)
    art = _sc_article(ids, article_emb)
    return _tc_dense(
        art, sidx, section_emb, group_emb, graphical_emb, colour_emb,
        gamma.reshape(1, 128), beta.reshape(1, 128),
        moving_mean.reshape(1, 128), moving_var.reshape(1, 128), W)
